# baseline (speedup 1.0000x reference)
import functools

import jax
import jax.numpy as jnp
from jax import lax
from jax.experimental import pallas as pl
from jax.experimental.pallas import tpu as pltpu
from jax.experimental.pallas import tpu_sc as plsc


def kernel(out_temp, out, coord):
    N, C, H, W = out.shape
    _, _, th, tw = out_temp.shape
    M = N * C
    temp3 = out_temp.reshape(M, th, tw)
    out3 = out.reshape(M, H, W)

    NW = 32
    per = M // NW
    RC = 64
    npi = H // RC
    nsteps = per * npi
    NBUF = 3
    LA = 2

    mesh = plsc.VectorSubcoreMesh(core_axis_name="c", subcore_axis_name="s")

    @functools.partial(
        pl.kernel,
        out_type=jax.ShapeDtypeStruct((M, H, W), out.dtype),
        mesh=mesh,
        scratch_types=[
            pltpu.VMEM((16,), jnp.int32),
            pltpu.VMEM_SHARED((16, NBUF, RC, W), jnp.float32),
            pltpu.SemaphoreType.DMA((NBUF,)),
            pltpu.SemaphoreType.DMA((NBUF,)),
        ],
    )
    def k(temp_hbm, out_hbm, coord_hbm, res_hbm, cvec, shared, rsem, wsem):
        sid = lax.axis_index("s")
        bufs = [shared.at[sid, b] for b in range(NBUF)]
        wid = lax.axis_index("s") * 2 + lax.axis_index("c")
        pltpu.sync_copy(coord_hbm, cvec.at[pl.ds(0, 4)])
        cv = cvec[...]
        x0 = pl.multiple_of(cv[0], 128)
        y0 = pl.multiple_of(cv[2], 8)
        xr = pl.multiple_of(x0 + tw, 128)
        base = wid * per
        lw = (W - tw) // 2
        tr0 = y0 // RC

        def issue_reads(s):
            slot = s % NBUF
            j, c = divmod(s, npi)
            i = base + j
            buf = bufs[slot]
            sem = rsem.at[slot]
            cc = jnp.int32(c)
            inside = jnp.logical_and(cc >= tr0, cc < tr0 + th // RC)

            @pl.when(jnp.logical_not(inside))
            def _():
                hc = RC // 2
                pltpu.async_copy(out_hbm.at[i, pl.ds(c * RC, hc)],
                                 buf.at[pl.ds(0, hc)], sem)
                pltpu.async_copy(out_hbm.at[i, pl.ds(c * RC + hc, hc)],
                                 buf.at[pl.ds(hc, hc)], sem)

            @pl.when(inside)
            def _():
                r0 = pl.multiple_of(c * RC - y0, 8)
                pltpu.async_copy(
                    out_hbm.at[i, pl.ds(c * RC, RC), pl.ds(0, lw)],
                    buf.at[:, pl.ds(0, lw)], sem)
                pltpu.async_copy(
                    temp_hbm.at[i, pl.ds(r0, RC)],
                    buf.at[:, pl.ds(x0, tw)], sem)
                pltpu.async_copy(
                    out_hbm.at[i, pl.ds(c * RC, RC), pl.ds(xr, lw)],
                    buf.at[:, pl.ds(xr, lw)], sem)

        def wait_reads(s):
            slot = s % NBUF
            j, c = divmod(s, npi)
            i = base + j
            buf = bufs[slot]
            sem = rsem.at[slot]
            cc = jnp.int32(c)
            inside = jnp.logical_and(cc >= tr0, cc < tr0 + th // RC)

            @pl.when(jnp.logical_not(inside))
            def _():
                hc = RC // 2
                pltpu.make_async_copy(
                    out_hbm.at[i, pl.ds(c * RC, hc)],
                    buf.at[pl.ds(0, hc)], sem).wait()
                pltpu.make_async_copy(
                    out_hbm.at[i, pl.ds(c * RC + hc, hc)],
                    buf.at[pl.ds(hc, hc)], sem).wait()

            @pl.when(inside)
            def _():
                r0 = pl.multiple_of(c * RC - y0, 8)
                pltpu.make_async_copy(
                    out_hbm.at[i, pl.ds(c * RC, RC), pl.ds(0, lw)],
                    buf.at[:, pl.ds(0, lw)], sem).wait()
                pltpu.make_async_copy(
                    temp_hbm.at[i, pl.ds(r0, RC)],
                    buf.at[:, pl.ds(x0, tw)], sem).wait()
                pltpu.make_async_copy(
                    out_hbm.at[i, pl.ds(c * RC, RC), pl.ds(xr, lw)],
                    buf.at[:, pl.ds(xr, lw)], sem).wait()

        def write_copy(s):
            slot = s % NBUF
            j, c = divmod(s, npi)
            i = base + j
            return pltpu.make_async_copy(
                bufs[slot], res_hbm.at[i, pl.ds(c * RC, RC)], wsem.at[slot])

        for s in range(min(LA, nsteps)):
            issue_reads(s)
        for s in range(nsteps):
            wait_reads(s)
            write_copy(s).start()
            nxt = s + LA
            if nxt < nsteps:
                if nxt >= NBUF:
                    write_copy(nxt - NBUF).wait()
                issue_reads(nxt)
        for s in range(max(nsteps - NBUF, 0), nsteps):
            write_copy(s).wait()

    res = k(temp3, out3, coord)
    return res.reshape(N, C, H, W)


# speedup vs baseline: 1.0066x; 1.0066x over previous
import functools

import jax
import jax.numpy as jnp
from jax import lax
from jax.experimental import pallas as pl
from jax.experimental.pallas import tpu as pltpu
from jax.experimental.pallas import tpu_sc as plsc


def kernel(out_temp, out, coord):
    N, C, H, W = out.shape
    _, _, th, tw = out_temp.shape
    M = N * C
    temp3 = out_temp.reshape(M, th, tw)
    out3 = out.reshape(M, H, W)

    NW = 32
    per = M // NW
    RC = 64
    npi = H // RC
    nsteps = per * npi
    _corder = [0, 2, 6, 3, 1, 4, 7, 5]
    NBUF = 3
    LA = 2

    mesh = plsc.VectorSubcoreMesh(core_axis_name="c", subcore_axis_name="s")

    @functools.partial(
        pl.kernel,
        out_type=jax.ShapeDtypeStruct((M, H, W), out.dtype),
        mesh=mesh,
        scratch_types=[
            pltpu.VMEM((16,), jnp.int32),
            pltpu.VMEM_SHARED((16, NBUF, RC, W), jnp.float32),
            pltpu.SemaphoreType.DMA((NBUF,)),
            pltpu.SemaphoreType.DMA((NBUF,)),
        ],
    )
    def k(temp_hbm, out_hbm, coord_hbm, res_hbm, cvec, shared, rsem, wsem):
        sid = lax.axis_index("s")
        bufs = [shared.at[sid, b] for b in range(NBUF)]
        wid = lax.axis_index("s") * 2 + lax.axis_index("c")
        pltpu.sync_copy(coord_hbm, cvec.at[pl.ds(0, 4)])
        cv = cvec[...]
        x0 = pl.multiple_of(cv[0], 128)
        y0 = pl.multiple_of(cv[2], 8)
        xr = pl.multiple_of(x0 + tw, 128)
        base = wid * per
        lw = (W - tw) // 2
        tr0 = y0 // RC

        def issue_reads(s):
            slot = s % NBUF
            j, c = divmod(s, npi)
            c = _corder[c]
            i = base + j
            buf = bufs[slot]
            sem = rsem.at[slot]
            cc = jnp.int32(c)
            inside = jnp.logical_and(cc >= tr0, cc < tr0 + th // RC)

            @pl.when(jnp.logical_not(inside))
            def _():
                pltpu.async_copy(out_hbm.at[i, pl.ds(c * RC, RC)], buf, sem)

            @pl.when(inside)
            def _():
                r0 = pl.multiple_of(c * RC - y0, 8)
                pltpu.async_copy(
                    out_hbm.at[i, pl.ds(c * RC, RC), pl.ds(0, lw)],
                    buf.at[:, pl.ds(0, lw)], sem)
                pltpu.async_copy(
                    temp_hbm.at[i, pl.ds(r0, RC)],
                    buf.at[:, pl.ds(x0, tw)], sem)
                pltpu.async_copy(
                    out_hbm.at[i, pl.ds(c * RC, RC), pl.ds(xr, lw)],
                    buf.at[:, pl.ds(xr, lw)], sem)

        def wait_reads(s):
            slot = s % NBUF
            j, c = divmod(s, npi)
            c = _corder[c]
            i = base + j
            buf = bufs[slot]
            sem = rsem.at[slot]
            cc = jnp.int32(c)
            inside = jnp.logical_and(cc >= tr0, cc < tr0 + th // RC)

            @pl.when(jnp.logical_not(inside))
            def _():
                pltpu.make_async_copy(
                    out_hbm.at[i, pl.ds(c * RC, RC)], buf, sem).wait()

            @pl.when(inside)
            def _():
                r0 = pl.multiple_of(c * RC - y0, 8)
                pltpu.make_async_copy(
                    out_hbm.at[i, pl.ds(c * RC, RC), pl.ds(0, lw)],
                    buf.at[:, pl.ds(0, lw)], sem).wait()
                pltpu.make_async_copy(
                    temp_hbm.at[i, pl.ds(r0, RC)],
                    buf.at[:, pl.ds(x0, tw)], sem).wait()
                pltpu.make_async_copy(
                    out_hbm.at[i, pl.ds(c * RC, RC), pl.ds(xr, lw)],
                    buf.at[:, pl.ds(xr, lw)], sem).wait()

        def write_copy(s):
            slot = s % NBUF
            j, c = divmod(s, npi)
            c = _corder[c]
            i = base + j
            return pltpu.make_async_copy(
                bufs[slot], res_hbm.at[i, pl.ds(c * RC, RC)], wsem.at[slot])

        for s in range(min(LA, nsteps)):
            issue_reads(s)
        for s in range(nsteps):
            wait_reads(s)
            write_copy(s).start()
            nxt = s + LA
            if nxt < nsteps:
                if nxt >= NBUF:
                    write_copy(nxt - NBUF).wait()
                issue_reads(nxt)
        for s in range(max(nsteps - NBUF, 0), nsteps):
            write_copy(s).wait()

    res = k(temp3, out3, coord)
    return res.reshape(N, C, H, W)


# speedup vs baseline: 1.0071x; 1.0005x over previous
import functools

import jax
import jax.numpy as jnp
from jax import lax
from jax.experimental import pallas as pl
from jax.experimental.pallas import tpu as pltpu
from jax.experimental.pallas import tpu_sc as plsc


def kernel(out_temp, out, coord):
    N, C, H, W = out.shape
    _, _, th, tw = out_temp.shape
    M = N * C
    temp3 = out_temp.reshape(M, th, tw)
    out3 = out.reshape(M, H, W)

    NW = 32
    per = M // NW
    RC = 64
    npi = H // RC
    nsteps = per * npi
    NBUF = 3
    LA = 2

    mesh = plsc.VectorSubcoreMesh(core_axis_name="c", subcore_axis_name="s")

    @functools.partial(
        pl.kernel,
        out_type=jax.ShapeDtypeStruct((M, H, W), out.dtype),
        mesh=mesh,
        scratch_types=[
            pltpu.VMEM((16,), jnp.int32),
            pltpu.VMEM_SHARED((16, NBUF, RC, W), jnp.float32),
            pltpu.SemaphoreType.DMA((NBUF,)),
            pltpu.SemaphoreType.DMA((NBUF,)),
        ],
    )
    def k(temp_hbm, out_hbm, coord_hbm, res_hbm, cvec, shared, rsem, wsem):
        sid = lax.axis_index("s")
        bufs = [shared.at[sid, b] for b in range(NBUF)]
        wid = lax.axis_index("s") * 2 + lax.axis_index("c")
        pltpu.sync_copy(coord_hbm, cvec.at[pl.ds(0, 4)])
        cv = cvec[...]
        x0 = pl.multiple_of(cv[0], 128)
        y0 = pl.multiple_of(cv[2], 8)
        xr = pl.multiple_of(x0 + tw, 128)
        base = wid * per
        lw = (W - tw) // 2
        tr0 = y0 // RC

        def issue_reads(s):
            slot = s % NBUF
            j, c = divmod(s, npi)
            i = base + j
            buf = bufs[slot]
            sem = rsem.at[slot]
            cc = jnp.int32(c)
            inside = jnp.logical_and(cc >= tr0, cc < tr0 + th // RC)

            @pl.when(jnp.logical_not(inside))
            def _():
                pltpu.async_copy(out_hbm.at[i, pl.ds(c * RC, RC)], buf, sem)

            @pl.when(inside)
            def _():
                r0 = pl.multiple_of(c * RC - y0, 8)
                pltpu.async_copy(
                    out_hbm.at[i, pl.ds(c * RC, RC), pl.ds(0, lw)],
                    buf.at[:, pl.ds(0, lw)], sem)
                pltpu.async_copy(
                    temp_hbm.at[i, pl.ds(r0, RC)],
                    buf.at[:, pl.ds(x0, tw)], sem)
                pltpu.async_copy(
                    out_hbm.at[i, pl.ds(c * RC, RC), pl.ds(xr, lw)],
                    buf.at[:, pl.ds(xr, lw)], sem)

        def wait_reads(s):
            slot = s % NBUF
            j, c = divmod(s, npi)
            i = base + j
            buf = bufs[slot]
            sem = rsem.at[slot]
            cc = jnp.int32(c)
            inside = jnp.logical_and(cc >= tr0, cc < tr0 + th // RC)

            @pl.when(jnp.logical_not(inside))
            def _():
                pltpu.make_async_copy(
                    out_hbm.at[i, pl.ds(c * RC, RC)], buf, sem).wait()

            @pl.when(inside)
            def _():
                r0 = pl.multiple_of(c * RC - y0, 8)
                pltpu.make_async_copy(
                    out_hbm.at[i, pl.ds(c * RC, RC), pl.ds(0, lw)],
                    buf.at[:, pl.ds(0, lw)], sem).wait()
                pltpu.make_async_copy(
                    temp_hbm.at[i, pl.ds(r0, RC)],
                    buf.at[:, pl.ds(x0, tw)], sem).wait()
                pltpu.make_async_copy(
                    out_hbm.at[i, pl.ds(c * RC, RC), pl.ds(xr, lw)],
                    buf.at[:, pl.ds(xr, lw)], sem).wait()

        def write_copy(s):
            slot = s % NBUF
            j, c = divmod(s, npi)
            i = base + j
            return pltpu.make_async_copy(
                bufs[slot], res_hbm.at[i, pl.ds(c * RC, RC)], wsem.at[slot])

        for s in range(min(LA, nsteps)):
            issue_reads(s)
        for s in range(nsteps):
            wait_reads(s)
            write_copy(s).start()
            nxt = s + LA
            if nxt < nsteps:
                if nxt >= NBUF:
                    write_copy(nxt - NBUF).wait()
                issue_reads(nxt)
        for s in range(max(nsteps - NBUF, 0), nsteps):
            write_copy(s).wait()

    res = k(temp3, out3, coord)
    return res.reshape(N, C, H, W)
